# Initial kernel scaffold; baseline (speedup 1.0000x reference)
import jax
import jax.numpy as jnp
from jax.experimental import pallas as pl


def kernel(node_feature, edge_src, edge_dst):
    raise NotImplementedError("write your pallas kernel here")


# speedup vs baseline: 1.9872x; 1.9872x over previous
import functools

import jax
import jax.numpy as jnp
from jax import lax
from jax.experimental import pallas as pl
from jax.experimental.pallas import tpu as pltpu
from jax.experimental.pallas import tpu_sc as plsc

N_NODES = 10000
N_EDGES = 320000
D_FEAT_DIM = 128

NUM_ROWS = 2 * N_EDGES
CHUNK = 80
NC, NS = 2, 16
NW = NC * NS
ROWS_PER_W = NUM_ROWS // NW
NCHUNK = ROWS_PER_W // CHUNK

_mesh = plsc.VectorSubcoreMesh(core_axis_name="c", subcore_axis_name="s")


@functools.partial(
    pl.kernel,
    mesh=_mesh,
    out_type=jax.ShapeDtypeStruct((NUM_ROWS, D_FEAT_DIM), jnp.float32),
    scratch_types=[
        pltpu.VMEM((NCHUNK, CHUNK), jnp.int32),
        pltpu.VMEM((CHUNK, D_FEAT_DIM), jnp.float32),
        pltpu.VMEM((CHUNK, D_FEAT_DIM), jnp.float32),
        pltpu.SemaphoreType.DMA,
        pltpu.SemaphoreType.DMA,
        pltpu.SemaphoreType.DMA,
        pltpu.SemaphoreType.DMA,
    ],
)
def _gather_rows(table, idx2d, out, idx_v, rows0, rows1, g0, g1, w0, w1):
    wid = lax.axis_index("s") * NC + lax.axis_index("c")
    chunk0 = wid * NCHUNK
    pltpu.sync_copy(idx2d.at[wid], idx_v)

    pltpu.make_async_copy(table.at[idx_v.at[0]], rows0, g0).start()

    def body(i, _):
        j0 = 2 * i
        j1 = 2 * i + 1
        pltpu.make_async_copy(table.at[idx_v.at[j0]], rows0, g0).wait()
        pltpu.make_async_copy(
            rows0, out.at[pl.ds((chunk0 + j0) * CHUNK, CHUNK)], w0
        ).start()
        @pl.when(i >= 1)
        def _():
            pltpu.make_async_copy(rows1, out.at[pl.ds(0, CHUNK)], w1).wait()

        pltpu.make_async_copy(table.at[idx_v.at[j1]], rows1, g1).start()
        pltpu.make_async_copy(table.at[idx_v.at[j1]], rows1, g1).wait()
        pltpu.make_async_copy(
            rows1, out.at[pl.ds((chunk0 + j1) * CHUNK, CHUNK)], w1
        ).start()
        pltpu.make_async_copy(rows0, out.at[pl.ds(0, CHUNK)], w0).wait()

        @pl.when(i + 1 < NCHUNK // 2)
        def _():
            pltpu.make_async_copy(table.at[idx_v.at[j1 + 1]], rows0, g0).start()

        return 0

    lax.fori_loop(0, NCHUNK // 2, body, 0)

    pltpu.make_async_copy(rows1, out.at[pl.ds(0, CHUNK)], w1).wait()


def kernel(node_feature, edge_src, edge_dst):
    inter = jnp.stack(
        [edge_src.astype(jnp.int32), edge_dst.astype(jnp.int32)], axis=1
    ).reshape(NW, NCHUNK, CHUNK)
    out = _gather_rows(node_feature, inter)
    return out.reshape(N_EDGES, 2 * D_FEAT_DIM)


# speedup vs baseline: 2.2718x; 1.1432x over previous
import functools

import jax
import jax.numpy as jnp
from jax import lax
from jax.experimental import pallas as pl
from jax.experimental.pallas import tpu as pltpu
from jax.experimental.pallas import tpu_sc as plsc

N_NODES = 10000
N_EDGES = 320000
D_FEAT_DIM = 128

NUM_ROWS = 2 * N_EDGES
CHUNK = 80
NC, NS = 2, 16
NW = NC * NS
ROWS_PER_W = NUM_ROWS // NW
NCHUNK = ROWS_PER_W // CHUNK
S = 5
G = 3

_mesh = plsc.VectorSubcoreMesh(core_axis_name="c", subcore_axis_name="s")


@functools.partial(
    pl.kernel,
    mesh=_mesh,
    out_type=jax.ShapeDtypeStruct((NUM_ROWS, D_FEAT_DIM), jnp.float32),
    scratch_types=[
        pltpu.VMEM((NCHUNK, CHUNK), jnp.int32),
        pltpu.VMEM((S, CHUNK, D_FEAT_DIM), jnp.float32),
    ]
    + [pltpu.SemaphoreType.DMA] * (2 * S),
)
def _gather_rows(table, idx2d, out, idx_v, rows, *sems):
    gsem = sems[:S]
    wsem = sems[S:]
    wid = lax.axis_index("s") * NC + lax.axis_index("c")
    chunk0 = wid * NCHUNK
    pltpu.sync_copy(idx2d.at[wid], idx_v)

    for k in range(G):
        pltpu.make_async_copy(table.at[idx_v.at[k]], rows.at[k], gsem[k]).start()

    def body(i, _):
        for s in range(S):
            j = S * i + s
            sn = (s + G) % S
            pltpu.make_async_copy(table.at[idx_v.at[j]], rows.at[s], gsem[s]).wait()
            pltpu.make_async_copy(
                rows.at[s], out.at[pl.ds((chunk0 + j) * CHUNK, CHUNK)], wsem[s]
            ).start()
            @pl.when(j >= S - G)
            def _():
                pltpu.make_async_copy(
                    rows.at[sn], out.at[pl.ds(0, CHUNK)], wsem[sn]
                ).wait()

            @pl.when(j + G < NCHUNK)
            def _():
                pltpu.make_async_copy(
                    table.at[idx_v.at[j + G]], rows.at[sn], gsem[sn]
                ).start()

        return 0

    lax.fori_loop(0, NCHUNK // S, body, 0)

    for j in range(NCHUNK - (S - G), NCHUNK):
        s = j % S
        pltpu.make_async_copy(
            rows.at[s], out.at[pl.ds(0, CHUNK)], wsem[s]
        ).wait()


def kernel(node_feature, edge_src, edge_dst):
    inter = jnp.stack(
        [edge_src.astype(jnp.int32), edge_dst.astype(jnp.int32)], axis=1
    ).reshape(NW, NCHUNK, CHUNK)
    out = _gather_rows(node_feature, inter)
    return out.reshape(N_EDGES, 2 * D_FEAT_DIM)


# speedup vs baseline: 7.3614x; 3.2404x over previous
import functools

import jax
import jax.numpy as jnp
from jax import lax
from jax.experimental import pallas as pl
from jax.experimental.pallas import tpu as pltpu
from jax.experimental.pallas import tpu_sc as plsc

N_NODES = 10000
N_EDGES = 320000
D_FEAT_DIM = 128

NC, NS = 2, 16
NW = NC * NS
CHUNK = 80
ROWS_PER_W = 2 * N_EDGES // NW
NCHUNK = ROWS_PER_W // CHUNK
S = 5
G = 3

_mesh = plsc.VectorSubcoreMesh(core_axis_name="c", subcore_axis_name="s")


@functools.partial(
    pl.kernel,
    mesh=_mesh,
    out_type=jax.ShapeDtypeStruct((N_EDGES, 2 * D_FEAT_DIM), jnp.float32),
    scratch_types=[
        pltpu.VMEM((ROWS_PER_W,), jnp.int32),
        pltpu.VMEM((S, CHUNK, D_FEAT_DIM), jnp.float32),
    ]
    + [pltpu.SemaphoreType.DMA] * (2 * S),
)
def _gather_rows(table, idx_all, out, idx_v, rows, *sems):
    gsem = sems[:S]
    wsem = sems[S:]
    wid = lax.axis_index("s") * NC + lax.axis_index("c")
    col = (wid // NS) * D_FEAT_DIM
    ebase = (wid % NS) * ROWS_PER_W
    pltpu.sync_copy(idx_all.at[pl.ds(wid * ROWS_PER_W, ROWS_PER_W)], idx_v)

    def gather_start(j, s):
        pltpu.make_async_copy(
            table.at[idx_v.at[pl.ds(j * CHUNK, CHUNK)]], rows.at[s], gsem[s]
        ).start()

    def gather_wait(j, s):
        pltpu.make_async_copy(
            table.at[idx_v.at[pl.ds(j * CHUNK, CHUNK)]], rows.at[s], gsem[s]
        ).wait()

    def write_start(j, s):
        pltpu.make_async_copy(
            rows.at[s],
            out.at[pl.ds(ebase + j * CHUNK, CHUNK), pl.ds(col, D_FEAT_DIM)],
            wsem[s],
        ).start()

    def write_wait(s):
        pltpu.make_async_copy(
            rows.at[s],
            out.at[pl.ds(0, CHUNK), pl.ds(0, D_FEAT_DIM)],
            wsem[s],
        ).wait()

    for k in range(G):
        gather_start(k, k)

    def body(i, _):
        for s in range(S):
            j = S * i + s
            sn = (s + G) % S
            gather_wait(j, s)
            write_start(j, s)
            @pl.when(j >= S - G)
            def _():
                write_wait(sn)

            @pl.when(j + G < NCHUNK)
            def _():
                gather_start(j + G, sn)

        return 0

    lax.fori_loop(0, NCHUNK // S, body, 0)

    for j in range(NCHUNK - (S - G), NCHUNK):
        write_wait(j % S)


def kernel(node_feature, edge_src, edge_dst):
    idx_all = jnp.concatenate(
        [edge_src.astype(jnp.int32), edge_dst.astype(jnp.int32)]
    )
    return _gather_rows(node_feature, idx_all)


# speedup vs baseline: 7.3912x; 1.0040x over previous
import functools

import jax
import jax.numpy as jnp
from jax import lax
from jax.experimental import pallas as pl
from jax.experimental.pallas import tpu as pltpu
from jax.experimental.pallas import tpu_sc as plsc

N_NODES = 10000
N_EDGES = 320000
D_FEAT_DIM = 128

NC, NS = 2, 16
NW = NC * NS
CHUNK = 128
ROWS_PER_W = 2 * N_EDGES // NW
NFULL = ROWS_PER_W // CHUNK
TAIL = ROWS_PER_W - NFULL * CHUNK
S = 6
G = 3

_mesh = plsc.VectorSubcoreMesh(core_axis_name="c", subcore_axis_name="s")


@functools.partial(
    pl.kernel,
    mesh=_mesh,
    out_type=jax.ShapeDtypeStruct((N_EDGES, 2 * D_FEAT_DIM), jnp.float32),
    scratch_types=[
        pltpu.VMEM((ROWS_PER_W,), jnp.int32),
        pltpu.VMEM((S, CHUNK, D_FEAT_DIM), jnp.float32),
    ]
    + [pltpu.SemaphoreType.DMA] * (2 * S),
)
def _gather_rows(table, idx_all, out, idx_v, rows, *sems):
    gsem = sems[:S]
    wsem = sems[S:]
    wid = lax.axis_index("s") * NC + lax.axis_index("c")
    col = (wid // NS) * D_FEAT_DIM
    ebase = (wid % NS) * ROWS_PER_W
    pltpu.sync_copy(idx_all.at[pl.ds(wid * ROWS_PER_W, ROWS_PER_W)], idx_v)

    def gather_start(j, s):
        pltpu.make_async_copy(
            table.at[idx_v.at[pl.ds(j * CHUNK, CHUNK)]], rows.at[s], gsem[s]
        ).start()

    def gather_wait(j, s):
        pltpu.make_async_copy(
            table.at[idx_v.at[pl.ds(j * CHUNK, CHUNK)]], rows.at[s], gsem[s]
        ).wait()

    def write_start(j, s):
        pltpu.make_async_copy(
            rows.at[s],
            out.at[pl.ds(ebase + j * CHUNK, CHUNK), pl.ds(col, D_FEAT_DIM)],
            wsem[s],
        ).start()

    def write_wait(s):
        pltpu.make_async_copy(
            rows.at[s],
            out.at[pl.ds(0, CHUNK), pl.ds(0, D_FEAT_DIM)],
            wsem[s],
        ).wait()

    for k in range(G):
        gather_start(k, k)

    def body(i, _):
        for s in range(S):
            j = S * i + s
            sn = (s + G) % S
            gather_wait(j, s)
            write_start(j, s)
            @pl.when(j >= S - G)
            def _():
                write_wait(sn)

            @pl.when(j + G < NFULL)
            def _():
                gather_start(j + G, sn)

        return 0

    lax.fori_loop(0, NFULL // S, body, 0)

    for j in range(NFULL - (S - G), NFULL):
        write_wait(j % S)

    pltpu.make_async_copy(
        table.at[idx_v.at[pl.ds(NFULL * CHUNK, TAIL)]],
        rows.at[0].at[pl.ds(0, TAIL)],
        gsem[0],
    ).start()
    pltpu.make_async_copy(
        table.at[idx_v.at[pl.ds(NFULL * CHUNK, TAIL)]],
        rows.at[0].at[pl.ds(0, TAIL)],
        gsem[0],
    ).wait()
    pltpu.sync_copy(
        rows.at[0].at[pl.ds(0, TAIL)],
        out.at[pl.ds(ebase + NFULL * CHUNK, TAIL), pl.ds(col, D_FEAT_DIM)],
    )


def kernel(node_feature, edge_src, edge_dst):
    idx_all = jnp.concatenate(
        [edge_src.astype(jnp.int32), edge_dst.astype(jnp.int32)]
    )
    return _gather_rows(node_feature, idx_all)


# speedup vs baseline: 12.3503x; 1.6709x over previous
import functools

import jax
import jax.numpy as jnp
from jax import lax
from jax.experimental import pallas as pl
from jax.experimental.pallas import tpu as pltpu
from jax.experimental.pallas import tpu_sc as plsc

N_NODES = 10000
N_EDGES = 320000
D_FEAT_DIM = 128

NC, NS = 2, 16
NW = NC * NS
CHUNK = 48
ROWS_PER_W = 2 * N_EDGES // NW
NFULL = ROWS_PER_W // CHUNK
TAIL = ROWS_PER_W - NFULL * CHUNK
S = 4
G = 2

_mesh = plsc.VectorSubcoreMesh(core_axis_name="c", subcore_axis_name="s")


@functools.partial(
    pl.kernel,
    mesh=_mesh,
    out_type=jax.ShapeDtypeStruct((N_EDGES, 2 * D_FEAT_DIM), jnp.float32),
    scratch_types=[
        pltpu.VMEM((ROWS_PER_W,), jnp.int32),
        pltpu.VMEM((S, CHUNK, D_FEAT_DIM), jnp.float32),
        pltpu.VMEM_SHARED((N_NODES, D_FEAT_DIM), jnp.float32),
    ]
    + [pltpu.SemaphoreType.DMA] * (2 * S),
)
def _gather_rows(table, idx_all, out, idx_v, rows, table_sp, *sems):
    gsem = sems[:S]
    wsem = sems[S:]
    wid = lax.axis_index("s") * NC + lax.axis_index("c")
    sub = lax.axis_index("s")
    col = (wid // NS) * D_FEAT_DIM
    ebase = (wid % NS) * ROWS_PER_W

    STG = 624
    pltpu.sync_copy(
        table.at[pl.ds(sub * STG, STG)], table_sp.at[pl.ds(sub * STG, STG)]
    )

    @pl.when(sub == 0)
    def _():
        pltpu.sync_copy(
            table.at[pl.ds(NS * STG, N_NODES - NS * STG)],
            table_sp.at[pl.ds(NS * STG, N_NODES - NS * STG)],
        )

    pltpu.sync_copy(idx_all.at[pl.ds(wid * ROWS_PER_W, ROWS_PER_W)], idx_v)
    plsc.subcore_barrier()

    def gather_start(j, s):
        pltpu.make_async_copy(
            table_sp.at[idx_v.at[pl.ds(j * CHUNK, CHUNK)]], rows.at[s], gsem[s]
        ).start()

    def gather_wait(j, s):
        pltpu.make_async_copy(
            table_sp.at[idx_v.at[pl.ds(j * CHUNK, CHUNK)]], rows.at[s], gsem[s]
        ).wait()

    def write_start(j, s):
        pltpu.make_async_copy(
            rows.at[s],
            out.at[pl.ds(ebase + j * CHUNK, CHUNK), pl.ds(col, D_FEAT_DIM)],
            wsem[s],
        ).start()

    def write_wait(s):
        pltpu.make_async_copy(
            rows.at[s],
            out.at[pl.ds(0, CHUNK), pl.ds(0, D_FEAT_DIM)],
            wsem[s],
        ).wait()

    for k in range(G):
        gather_start(k, k)

    def body(i, _):
        for s in range(S):
            j = S * i + s
            sn = (s + G) % S
            gather_wait(j, s)
            write_start(j, s)
            @pl.when(j >= S - G)
            def _():
                write_wait(sn)

            @pl.when(j + G < NFULL)
            def _():
                gather_start(j + G, sn)

        return 0

    lax.fori_loop(0, NFULL // S, body, 0)

    for j in range(NFULL - (S - G), NFULL):
        write_wait(j % S)

    pltpu.make_async_copy(
        table_sp.at[idx_v.at[pl.ds(NFULL * CHUNK, TAIL)]],
        rows.at[0].at[pl.ds(0, TAIL)],
        gsem[0],
    ).start()
    pltpu.make_async_copy(
        table_sp.at[idx_v.at[pl.ds(NFULL * CHUNK, TAIL)]],
        rows.at[0].at[pl.ds(0, TAIL)],
        gsem[0],
    ).wait()
    pltpu.sync_copy(
        rows.at[0].at[pl.ds(0, TAIL)],
        out.at[pl.ds(ebase + NFULL * CHUNK, TAIL), pl.ds(col, D_FEAT_DIM)],
    )


def kernel(node_feature, edge_src, edge_dst):
    idx_all = jnp.concatenate(
        [edge_src.astype(jnp.int32), edge_dst.astype(jnp.int32)]
    )
    return _gather_rows(node_feature, idx_all)


# speedup vs baseline: 12.6245x; 1.0222x over previous
import functools

import jax
import jax.numpy as jnp
from jax import lax
from jax.experimental import pallas as pl
from jax.experimental.pallas import tpu as pltpu
from jax.experimental.pallas import tpu_sc as plsc

N_NODES = 10000
N_EDGES = 320000
D_FEAT_DIM = 128

NC, NS = 2, 16
NW = NC * NS
CHUNK = 40
ROWS_PER_W = 2 * N_EDGES // NW
NFULL = ROWS_PER_W // CHUNK
TAIL = ROWS_PER_W - NFULL * CHUNK
S = 5
G = 3

_mesh = plsc.VectorSubcoreMesh(core_axis_name="c", subcore_axis_name="s")


@functools.partial(
    pl.kernel,
    mesh=_mesh,
    out_type=jax.ShapeDtypeStruct((N_EDGES, 2 * D_FEAT_DIM), jnp.float32),
    scratch_types=[
        pltpu.VMEM((ROWS_PER_W,), jnp.int32),
        pltpu.VMEM((S, CHUNK, D_FEAT_DIM), jnp.float32),
        pltpu.VMEM_SHARED((N_NODES, D_FEAT_DIM), jnp.float32),
    ]
    + [pltpu.SemaphoreType.DMA] * (2 * S),
)
def _gather_rows(table, idx_all, out, idx_v, rows, table_sp, *sems):
    gsem = sems[:S]
    wsem = sems[S:]
    wid = lax.axis_index("s") * NC + lax.axis_index("c")
    sub = lax.axis_index("s")
    col = (wid // NS) * D_FEAT_DIM
    ebase = (wid % NS) * ROWS_PER_W

    STG = 624
    pltpu.sync_copy(
        table.at[pl.ds(sub * STG, STG)], table_sp.at[pl.ds(sub * STG, STG)]
    )

    @pl.when(sub == 0)
    def _():
        pltpu.sync_copy(
            table.at[pl.ds(NS * STG, N_NODES - NS * STG)],
            table_sp.at[pl.ds(NS * STG, N_NODES - NS * STG)],
        )

    pltpu.sync_copy(idx_all.at[pl.ds(wid * ROWS_PER_W, ROWS_PER_W)], idx_v)
    plsc.subcore_barrier()

    def gather_start(j, s):
        pltpu.make_async_copy(
            table_sp.at[idx_v.at[pl.ds(j * CHUNK, CHUNK)]], rows.at[s], gsem[s]
        ).start()

    def gather_wait(j, s):
        pltpu.make_async_copy(
            table_sp.at[idx_v.at[pl.ds(j * CHUNK, CHUNK)]], rows.at[s], gsem[s]
        ).wait()

    def write_start(j, s):
        pltpu.make_async_copy(
            rows.at[s],
            out.at[pl.ds(ebase + j * CHUNK, CHUNK), pl.ds(col, D_FEAT_DIM)],
            wsem[s],
        ).start()

    def write_wait(s):
        pltpu.make_async_copy(
            rows.at[s],
            out.at[pl.ds(0, CHUNK), pl.ds(0, D_FEAT_DIM)],
            wsem[s],
        ).wait()

    for k in range(G):
        gather_start(k, k)

    def body(i, _):
        for s in range(S):
            j = S * i + s
            sn = (s + G) % S
            gather_wait(j, s)
            write_start(j, s)
            @pl.when(j >= S - G)
            def _():
                write_wait(sn)

            @pl.when(j + G < NFULL)
            def _():
                gather_start(j + G, sn)

        return 0

    lax.fori_loop(0, NFULL // S, body, 0)

    for j in range(NFULL - (S - G), NFULL):
        write_wait(j % S)


def kernel(node_feature, edge_src, edge_dst):
    idx_all = jnp.concatenate(
        [edge_src.astype(jnp.int32), edge_dst.astype(jnp.int32)]
    )
    return _gather_rows(node_feature, idx_all)
